# baseline (speedup 1.0000x reference)
import functools

import jax
import jax.numpy as jnp
from jax import lax
from jax.experimental import pallas as pl
from jax.experimental.pallas import tpu as pltpu
from jax.experimental.pallas import tpu_sc as plsc

Nv_ = 100000
Nh_ = 16
BATCH_ = 4096
XMIN_ = -10.0
XMAX_ = 10.0
DX_ = (XMAX_ - XMIN_) / (Nv_ - 1)

_NC = 2
_NS = 16
_NW = _NC * _NS
_BPW = BATCH_ // _NW
_L = 16

_BS = 2048
_GRID = -(-Nv_ // _BS)


def _dense_body(b_ref, c_ref, wt_ref, p_ref):
    e = 1.0 + jnp.exp(c_ref[...] + wt_ref[...])
    while e.shape[0] > 1:
        m = e.shape[0] // 2
        e = e[:m] * e[m:]
    p_ref[...] = jnp.exp(b_ref[...]) * e[0]


_dense_tc = pl.pallas_call(
    _dense_body,
    grid=(_GRID,),
    in_specs=[
        pl.BlockSpec((_BS,), lambda i: (i,)),
        pl.BlockSpec((Nh_, 1), lambda i: (0, 0)),
        pl.BlockSpec((Nh_, _BS), lambda i: (0, i)),
    ],
    out_specs=pl.BlockSpec((_BS,), lambda i: (i,)),
    out_shape=jax.ShapeDtypeStruct((Nv_,), jnp.float32),
)


def _gather_body(x_hbm, p_hbm, out_hbm, x_v, idx_v, pv_v, sem):
    wid = lax.axis_index("s") * _NC + lax.axis_index("c")
    base = wid * _BPW

    pltpu.sync_copy(x_hbm.at[pl.ds(base, _BPW)], x_v)

    def idx_body(k, carry):
        off = pl.multiple_of(k * _L, _L)
        xv = x_v[pl.ds(off, _L)]
        ii = ((xv - XMIN_) / DX_).astype(jnp.int32)
        idx_v[pl.ds(off, _L)] = jnp.minimum(jnp.maximum(ii, 0), Nv_ - 1)
        return carry

    lax.fori_loop(0, _BPW // _L, idx_body, 0, unroll=False)

    pltpu.async_copy(p_hbm.at[idx_v], pv_v, sem).wait()
    pltpu.sync_copy(pv_v, out_hbm.at[pl.ds(base, _BPW)])


_gather_sc = functools.partial(
    pl.kernel,
    out_type=jax.ShapeDtypeStruct((BATCH_,), jnp.float32),
    mesh=plsc.VectorSubcoreMesh(core_axis_name="c", subcore_axis_name="s"),
    compiler_params=pltpu.CompilerParams(needs_layout_passes=False,
                                         use_tc_tiling_on_sc=False),
    scratch_types=[
        pltpu.VMEM((_BPW,), jnp.float32),
        pltpu.VMEM((_BPW,), jnp.int32),
        pltpu.VMEM((_BPW,), jnp.float32),
        pltpu.SemaphoreType.DMA,
    ],
)(_gather_body)


def kernel(x, b, c, w):
    p = _dense_tc(b, c[:, None], w.T)
    return _gather_sc(x, p)


# speedup vs baseline: 1.4793x; 1.4793x over previous
import functools

import jax
import jax.numpy as jnp
from jax import lax
from jax.experimental import pallas as pl
from jax.experimental.pallas import tpu as pltpu
from jax.experimental.pallas import tpu_sc as plsc

Nv_ = 100000
Nh_ = 16
BATCH_ = 4096
XMIN_ = -10.0
XMAX_ = 10.0
DX_ = (XMAX_ - XMIN_) / (Nv_ - 1)

_NC = 2
_NS = 16
_NW = _NC * _NS
_BPW = BATCH_ // _NW
_L = 16


def _idx_body_fn(x_hbm, b_hbm, idx2_hbm, eb_hbm,
                 x_v, idx2_v, b_v, eb_v, sem_b):
    wid = lax.axis_index("s") * _NC + lax.axis_index("c")
    base = wid * _BPW

    pltpu.sync_copy(x_hbm.at[pl.ds(base, _BPW)], x_v)

    def idx_body(k, carry):
        off = pl.multiple_of(k * _L, _L)
        xv = x_v[pl.ds(off, _L)]
        ii = ((xv - XMIN_) / DX_).astype(jnp.int32)
        ii = jnp.minimum(jnp.maximum(ii, 0), Nv_ - 1)
        for h in range(Nh_):
            idx2_v[h, pl.ds(off, _L)] = ii + h * Nv_
        return carry

    lax.fori_loop(0, _BPW // _L, idx_body, 0, unroll=False)

    cp_b = pltpu.async_copy(b_hbm.at[idx2_v.at[0]], b_v, sem_b)
    pltpu.sync_copy(idx2_v, idx2_hbm.at[wid])
    cp_b.wait()

    def eb_body(k, carry):
        off = pl.multiple_of(k * _L, _L)
        eb_v[pl.ds(off, _L)] = jnp.exp(b_v[pl.ds(off, _L)])
        return carry

    lax.fori_loop(0, _BPW // _L, eb_body, 0, unroll=False)
    pltpu.sync_copy(eb_v, eb_hbm.at[pl.ds(base, _BPW)])


_idx_sc = functools.partial(
    pl.kernel,
    out_type=(jax.ShapeDtypeStruct((_NW, Nh_, _BPW), jnp.int32),
              jax.ShapeDtypeStruct((BATCH_,), jnp.float32)),
    mesh=plsc.VectorSubcoreMesh(core_axis_name="c", subcore_axis_name="s"),
    compiler_params=pltpu.CompilerParams(needs_layout_passes=False,
                                         use_tc_tiling_on_sc=False),
    scratch_types=[
        pltpu.VMEM((_BPW,), jnp.float32),
        pltpu.VMEM((Nh_, _BPW), jnp.int32),
        pltpu.VMEM((_BPW,), jnp.float32),
        pltpu.VMEM((_BPW,), jnp.float32),
        pltpu.SemaphoreType.DMA,
    ],
)(_idx_body_fn)


def _gather_body_fn(idx2_hbm, eb_hbm, c_hbm, wt_hbm, out_hbm,
                    idx2_v, t_v, eb_v, c_v, ch_v, out_v, sem_w):
    wid = lax.axis_index("s") * _NC + lax.axis_index("c")
    base = wid * _BPW

    pltpu.sync_copy(idx2_hbm.at[wid], idx2_v)
    pltpu.sync_copy(eb_hbm.at[pl.ds(base, _BPW)], eb_v)
    pltpu.sync_copy(c_hbm, c_v)

    cps = [pltpu.async_copy(wt_hbm.at[idx2_v.at[h]], t_v.at[h], sem_w)
           for h in range(Nh_)]

    lane16 = lax.iota(jnp.int32, _L) * _L
    cv = c_v[...]
    for j in range(_L):
        plsc.store_scatter(ch_v, [lane16 + j], cv)

    for cp in cps:
        cp.wait()

    chs = [ch_v[pl.ds(h * _L, _L)] for h in range(Nh_)]

    def chunk_body(k, carry):
        off = pl.multiple_of(k * _L, _L)
        acc = eb_v[pl.ds(off, _L)]
        for h in range(Nh_):
            acc = acc * (1.0 + jnp.exp(chs[h] + t_v[h, pl.ds(off, _L)]))
        out_v[pl.ds(off, _L)] = acc
        return carry

    lax.fori_loop(0, _BPW // _L, chunk_body, 0, unroll=False)

    pltpu.sync_copy(out_v, out_hbm.at[pl.ds(base, _BPW)])


_gather_sc = functools.partial(
    pl.kernel,
    out_type=jax.ShapeDtypeStruct((BATCH_,), jnp.float32),
    mesh=plsc.VectorSubcoreMesh(core_axis_name="c", subcore_axis_name="s"),
    compiler_params=pltpu.CompilerParams(needs_layout_passes=False,
                                         use_tc_tiling_on_sc=False),
    scratch_types=[
        pltpu.VMEM((Nh_, _BPW), jnp.int32),
        pltpu.VMEM((Nh_, _BPW), jnp.float32),
        pltpu.VMEM((_BPW,), jnp.float32),
        pltpu.VMEM((Nh_,), jnp.float32),
        pltpu.VMEM((Nh_ * _L,), jnp.float32),
        pltpu.VMEM((_BPW,), jnp.float32),
        pltpu.SemaphoreType.DMA,
    ],
)(_gather_body_fn)


def kernel(x, b, c, w):
    idx2, eb = _idx_sc(x, b)
    return _gather_sc(idx2, eb, c, w.T.reshape(-1))


# speedup vs baseline: 1.5324x; 1.0359x over previous
import functools

import jax
import jax.numpy as jnp
from jax import lax
from jax.experimental import pallas as pl
from jax.experimental.pallas import tpu as pltpu
from jax.experimental.pallas import tpu_sc as plsc

Nv_ = 100000
Nh_ = 16
BATCH_ = 4096
XMIN_ = -10.0
XMAX_ = 10.0
DX_ = (XMAX_ - XMIN_) / (Nv_ - 1)

_NC = 2
_NS = 16
_NW = _NC * _NS
_BPW = BATCH_ // _NW
_L = 16


def _rbm_body(x_hbm, b_hbm, c_hbm, wt_hbm, out_hbm,
              x_v, idx2_v, t_v, b_v, c_v, ch_v, out_v, sem_w, sem_b):
    wid = lax.axis_index("s") * _NC + lax.axis_index("c")
    base = wid * _BPW

    pltpu.sync_copy(x_hbm.at[pl.ds(base, _BPW)], x_v)
    pltpu.sync_copy(c_hbm, c_v)

    def idx_body(k, carry):
        off = pl.multiple_of(k * _L, _L)
        xv = x_v[pl.ds(off, _L)]
        ii = ((xv - XMIN_) / DX_).astype(jnp.int32)
        ii = jnp.minimum(jnp.maximum(ii, 0), Nv_ - 1)
        for h in range(Nh_):
            idx2_v[h, pl.ds(off, _L)] = ii + h * Nv_
        return carry

    lax.fori_loop(0, _BPW // _L, idx_body, 0, unroll=False)

    cp_b = pltpu.async_copy(b_hbm.at[idx2_v.at[0]], b_v, sem_b)
    cps = [pltpu.async_copy(wt_hbm.at[idx2_v.at[h]], t_v.at[h], sem_w)
           for h in range(Nh_)]

    lane16 = lax.iota(jnp.int32, _L) * _L
    cv = c_v[...]
    for j in range(_L):
        plsc.store_scatter(ch_v, [lane16 + j], cv)

    chs = [ch_v[pl.ds(h * _L, _L)] for h in range(Nh_)]

    cp_b.wait()
    for cp in cps[:Nh_ // 2]:
        cp.wait()

    def half1_body(k, carry):
        off = pl.multiple_of(k * _L, _L)
        acc = jnp.exp(b_v[pl.ds(off, _L)])
        for h in range(Nh_ // 2):
            acc = acc * (1.0 + jnp.exp(chs[h] + t_v[h, pl.ds(off, _L)]))
        out_v[pl.ds(off, _L)] = acc
        return carry

    lax.fori_loop(0, _BPW // _L, half1_body, 0, unroll=False)

    for cp in cps[Nh_ // 2:]:
        cp.wait()

    def half2_body(k, carry):
        off = pl.multiple_of(k * _L, _L)
        acc = out_v[pl.ds(off, _L)]
        for h in range(Nh_ // 2, Nh_):
            acc = acc * (1.0 + jnp.exp(chs[h] + t_v[h, pl.ds(off, _L)]))
        out_v[pl.ds(off, _L)] = acc
        return carry

    lax.fori_loop(0, _BPW // _L, half2_body, 0, unroll=False)

    pltpu.sync_copy(out_v, out_hbm.at[pl.ds(base, _BPW)])


_SCRATCH = [
    pltpu.VMEM((_BPW,), jnp.float32),
    pltpu.VMEM((Nh_, _BPW), jnp.int32),
    pltpu.VMEM((Nh_, _BPW), jnp.float32),
    pltpu.VMEM((_BPW,), jnp.float32),
    pltpu.VMEM((Nh_,), jnp.float32),
    pltpu.VMEM((Nh_ * _L,), jnp.float32),
    pltpu.VMEM((_BPW,), jnp.float32),
    pltpu.SemaphoreType.DMA,
    pltpu.SemaphoreType.DMA,
]


def _prep(x, b, c, w):
    return x, b, c, w.T.reshape(-1)


_rbm_sc = functools.partial(
    pl.kernel,
    out_type=jax.ShapeDtypeStruct((BATCH_,), jnp.float32),
    mesh=plsc.VectorSubcoreMesh(core_axis_name="c", subcore_axis_name="s"),
    compiler_params=pltpu.CompilerParams(needs_layout_passes=False,
                                         use_tc_tiling_on_sc=False),
    scratch_types=_SCRATCH,
)(_rbm_body)


def kernel(x, b, c, w):
    return _rbm_sc(*_prep(x, b, c, w))


# speedup vs baseline: 1.5460x; 1.0089x over previous
import functools

import jax
import jax.numpy as jnp
from jax import lax
from jax.experimental import pallas as pl
from jax.experimental.pallas import tpu as pltpu
from jax.experimental.pallas import tpu_sc as plsc

Nv_ = 100000
Nh_ = 16
BATCH_ = 4096
XMIN_ = -10.0
XMAX_ = 10.0
DX_ = (XMAX_ - XMIN_) / (Nv_ - 1)

_NC = 2
_NS = 16
_NW = _NC * _NS
_BPW = BATCH_ // _NW
_L = 16


def _rbm_body(x_hbm, b_hbm, c_hbm, wt_hbm, out_hbm,
              x_v, idx2_v, t_v, b_v, c_v, ch_v, out_v, sem_w, sem_b):
    wid = lax.axis_index("s") * _NC + lax.axis_index("c")
    base = wid * _BPW

    pltpu.sync_copy(x_hbm.at[pl.ds(base, _BPW)], x_v)
    pltpu.sync_copy(c_hbm, c_v)

    def idx_body(k, carry):
        off = pl.multiple_of(k * _L, _L)
        xv = x_v[pl.ds(off, _L)]
        ii = ((xv - XMIN_) / DX_).astype(jnp.int32)
        ii = jnp.minimum(jnp.maximum(ii, 0), Nv_ - 1)
        for h in range(Nh_):
            idx2_v[h, pl.ds(off, _L)] = ii + h * Nv_
        return carry

    lax.fori_loop(0, _BPW // _L, idx_body, 0, unroll=False)

    cp_b = pltpu.async_copy(b_hbm.at[idx2_v.at[0]], b_v, sem_b)
    cps = [pltpu.async_copy(wt_hbm.at[idx2_v.at[h]], t_v.at[h], sem_w)
           for h in range(Nh_)]

    lane16 = lax.iota(jnp.int32, _L) * _L
    cv = c_v[...]
    for j in range(_L):
        plsc.store_scatter(ch_v, [lane16 + j], cv)

    cp_b.wait()
    for cp in cps:
        cp.wait()

    chs = [ch_v[pl.ds(h * _L, _L)] for h in range(Nh_)]

    def chunk_body(k, carry):
        off = pl.multiple_of(k * _L, _L)
        acc = jnp.exp(b_v[pl.ds(off, _L)])
        for h in range(Nh_):
            acc = acc * (1.0 + jnp.exp(chs[h] + t_v[h, pl.ds(off, _L)]))
        out_v[pl.ds(off, _L)] = acc
        return carry

    lax.fori_loop(0, _BPW // _L, chunk_body, 0, unroll=False)

    pltpu.sync_copy(out_v, out_hbm.at[pl.ds(base, _BPW)])


_SCRATCH = [
    pltpu.VMEM((_BPW,), jnp.float32),
    pltpu.VMEM((Nh_, _BPW), jnp.int32),
    pltpu.VMEM((Nh_, _BPW), jnp.float32),
    pltpu.VMEM((_BPW,), jnp.float32),
    pltpu.VMEM((Nh_,), jnp.float32),
    pltpu.VMEM((Nh_ * _L,), jnp.float32),
    pltpu.VMEM((_BPW,), jnp.float32),
    pltpu.SemaphoreType.DMA,
    pltpu.SemaphoreType.DMA,
]


def _prep(x, b, c, w):
    return x, b, c, w.T.reshape(-1)


_rbm_sc = functools.partial(
    pl.kernel,
    out_type=jax.ShapeDtypeStruct((BATCH_,), jnp.float32),
    mesh=plsc.VectorSubcoreMesh(core_axis_name="c", subcore_axis_name="s"),
    compiler_params=pltpu.CompilerParams(needs_layout_passes=False,
                                         use_tc_tiling_on_sc=False),
    scratch_types=_SCRATCH,
)(_rbm_body)


def kernel(x, b, c, w):
    return _rbm_sc(*_prep(x, b, c, w))
